# baseline (speedup 1.0000x reference)
import functools

import jax
import jax.numpy as jnp
from jax import lax
from jax.experimental import pallas as pl
from jax.experimental.pallas import tpu as pltpu
from jax.experimental.pallas import tpu_sc as plsc

_NC = 2
_NS = 16
_NW = _NC * _NS
_CHUNK = 128
_DEG_W = 8


def _sc_mesh():
    return plsc.VectorSubcoreMesh(core_axis_name="c", subcore_axis_name="s")


def _load_chunks(edge_hbm, row, idx_v, wid, n, fill, base, rem, kt):
    pltpu.sync_copy(edge_hbm.at[row].at[pl.ds(wid * base, base)],
                    idx_v.at[pl.ds(0, base)])
    for rr in range(base, kt):
        for t in range(_CHUNK // 16):
            idx_v[rr, pl.ds(16 * t, 16)] = jnp.full((16,), fill, jnp.int32)
    if rem:
        @pl.when(wid < rem)
        def _():
            pltpu.sync_copy(edge_hbm.at[row].at[pl.ds(base * _NW + wid, 1)],
                            idx_v.at[pl.ds(base, 1)])


def _make_deg_kernel(n, base, rem, kt, npad, rpt):
    @functools.partial(
        pl.kernel,
        out_type=jax.ShapeDtypeStruct((_NC, npad, _DEG_W), jnp.float32),
        mesh=_sc_mesh(),
        scratch_types=[
            pltpu.VMEM((kt, _CHUNK), jnp.int32),
            pltpu.VMEM((_CHUNK, _DEG_W), jnp.float32),
            pltpu.VMEM_SHARED((npad, _DEG_W), jnp.float32),
        ],
        compiler_params=pltpu.CompilerParams(use_tc_tiling_on_sc=False),
    )
    def deg_kernel(edge_hbm, ones_hbm, zeros_hbm, out_hbm, dst_v, ones_v, acc):
        c = lax.axis_index("c")
        s = lax.axis_index("s")
        wid = s * _NC + c
        row0 = s * rpt
        _load_chunks(edge_hbm, 1, dst_v, wid, n, n, base, rem, kt)
        pltpu.sync_copy(ones_hbm, ones_v)
        pltpu.sync_copy(zeros_hbm.at[pl.ds(row0, rpt)], acc.at[pl.ds(row0, rpt)])
        plsc.subcore_barrier()

        def body(j, carry):
            pltpu.sync_copy(ones_v, acc.at[dst_v.at[j]], add=True)
            return carry

        lax.fori_loop(0, kt, body, 0)
        plsc.subcore_barrier()
        pltpu.sync_copy(acc.at[pl.ds(row0, rpt)],
                        out_hbm.at[c].at[pl.ds(row0, rpt)])

    return deg_kernel


def _make_agg_kernel(n, base, rem, kt, npad, rpt, d):
    assert kt % 2 == 0 and kt >= 4

    @functools.partial(
        pl.kernel,
        out_type=jax.ShapeDtypeStruct((_NC, npad, d), jnp.float32),
        mesh=_sc_mesh(),
        scratch_types=[
            pltpu.VMEM((kt, _CHUNK), jnp.int32),
            pltpu.VMEM((kt, _CHUNK), jnp.int32),
            pltpu.VMEM((_CHUNK, d), jnp.float32),
            pltpu.VMEM((_CHUNK, d), jnp.float32),
            pltpu.VMEM_SHARED((npad, d), jnp.float32),
            pltpu.SemaphoreType.DMA,
            pltpu.SemaphoreType.DMA,
        ],
        compiler_params=pltpu.CompilerParams(use_tc_tiling_on_sc=False),
    )
    def agg_kernel(edge_hbm, table_hbm, zeros_hbm, out_hbm,
                   src_v, dst_v, rows_a, rows_b, acc, sem_a, sem_b):
        c = lax.axis_index("c")
        s = lax.axis_index("s")
        wid = s * _NC + c
        row0 = s * rpt
        _load_chunks(edge_hbm, 0, src_v, wid, n, 0, base, rem, kt)
        _load_chunks(edge_hbm, 1, dst_v, wid, n, n, base, rem, kt)
        pltpu.sync_copy(zeros_hbm.at[pl.ds(row0, rpt)], acc.at[pl.ds(row0, rpt)])
        plsc.subcore_barrier()

        def gather(j, buf, sem):
            pltpu.async_copy(table_hbm.at[src_v.at[j]], buf, sem)

        def drain(buf, sem):
            pltpu.make_async_copy(table_hbm.at[src_v.at[0]], buf, sem).wait()

        def scatter(j, buf):
            pltpu.sync_copy(buf, acc.at[dst_v.at[j]], add=True)

        gather(0, rows_a, sem_a)
        gather(1, rows_b, sem_b)

        def body(i, carry):
            j0 = 2 * i
            drain(rows_a, sem_a)
            scatter(j0, rows_a)
            gather(j0 + 2, rows_a, sem_a)
            drain(rows_b, sem_b)
            scatter(j0 + 1, rows_b)
            gather(j0 + 3, rows_b, sem_b)
            return carry

        lax.fori_loop(0, kt // 2 - 1, body, 0)
        drain(rows_a, sem_a)
        scatter(kt - 2, rows_a)
        drain(rows_b, sem_b)
        scatter(kt - 1, rows_b)
        plsc.subcore_barrier()
        pltpu.sync_copy(acc.at[pl.ds(row0, rpt)],
                        out_hbm.at[c].at[pl.ds(row0, rpt)])

    return agg_kernel


def _tc1a_body(x_ref, w_ref, praw_ref):
    praw_ref[...] = jnp.dot(x_ref[...], w_ref[...],
                            preferred_element_type=jnp.float32)


def _make_tc1b(n, d_hid, d1):
    def body(praw_ref, dparts_ref, pre_ref, r_ref):
        deg = dparts_ref[0, :n, :1] + dparts_ref[1, :n, :1] + 1.0
        r = lax.rsqrt(deg)
        r_ref[...] = r
        pre_ref[:, :d_hid] = praw_ref[...] * r
        pre_ref[:, d_hid:] = jnp.zeros((n, d1 - d_hid), jnp.float32)
    return body


def _make_tc2(n, d_hid, d_out, d2):
    def body(a1_ref, r_ref, w_ref, pre2_ref):
        r = r_ref[...]
        h = jnp.maximum((a1_ref[0, :n, :d_hid] + a1_ref[1, :n, :d_hid]) * r, 0.0)
        p = jnp.dot(h, w_ref[...], preferred_element_type=jnp.float32)
        pre2_ref[:, :d_out] = p * r
        pre2_ref[:, d_out:] = jnp.zeros((n, d2 - d_out), jnp.float32)
    return body


def _make_tc3(n, d_out):
    def body(a2_ref, r_ref, out_ref):
        out_ref[...] = (a2_ref[0, :n, :d_out] + a2_ref[1, :n, :d_out]) * r_ref[...]
    return body


def kernel(x, edge_index, W1, W2):
    n, d_feat = x.shape
    e = edge_index.shape[1]
    d_hid = W1.shape[1]
    d_out = W2.shape[1]

    d1 = 24
    d2 = 8
    npad = ((n + 127) // 128) * 128
    rpt = npad // _NS

    if e % _CHUNK:
        pad = _CHUNK - e % _CHUNK
        edge_index = jnp.concatenate(
            [edge_index,
             jnp.concatenate([jnp.zeros((1, pad), jnp.int32),
                              jnp.full((1, pad), n, jnp.int32)])], axis=1)
        e += pad
    nchunks = e // _CHUNK
    base = nchunks // _NW
    rem = nchunks % _NW
    kt = base + 2 - base % 2
    edge3 = edge_index.reshape(2, nchunks, _CHUNK)

    zeros_w = jnp.zeros((npad, _DEG_W), jnp.float32)
    zeros_d1 = jnp.zeros((npad, d1), jnp.float32)
    zeros_d2 = jnp.zeros((npad, d2), jnp.float32)
    ones_c = jnp.ones((_CHUNK, _DEG_W), jnp.float32)

    dparts = _make_deg_kernel(n, base, rem, kt, npad, rpt)(edge3, ones_c, zeros_w)

    praw = pl.pallas_call(
        _tc1a_body,
        out_shape=jax.ShapeDtypeStruct((n, d_hid), jnp.float32),
    )(x, W1)

    pre1s, r = pl.pallas_call(
        _make_tc1b(n, d_hid, d1),
        out_shape=[
            jax.ShapeDtypeStruct((n, d1), jnp.float32),
            jax.ShapeDtypeStruct((n, 1), jnp.float32),
        ],
    )(praw, dparts)

    a1 = _make_agg_kernel(n, base, rem, kt, npad, rpt, d1)(edge3, pre1s, zeros_d1)

    pre2s = pl.pallas_call(
        _make_tc2(n, d_hid, d_out, d2),
        out_shape=jax.ShapeDtypeStruct((n, d2), jnp.float32),
    )(a1, r, W2)

    a2 = _make_agg_kernel(n, base, rem, kt, npad, rpt, d2)(edge3, pre2s, zeros_d2)

    out = pl.pallas_call(
        _make_tc3(n, d_out),
        out_shape=jax.ShapeDtypeStruct((n, d_out), jnp.float32),
    )(a2, r)

    return out


# speedup vs baseline: 1.2026x; 1.2026x over previous
import functools

import jax
import jax.numpy as jnp
from jax import lax
from jax.experimental import pallas as pl
from jax.experimental.pallas import tpu as pltpu
from jax.experimental.pallas import tpu_sc as plsc

_NC = 2
_NS = 16
_NW = _NC * _NS
_CHUNK = 128
_DEG_W = 8


def _sc_mesh():
    return plsc.VectorSubcoreMesh(core_axis_name="c", subcore_axis_name="s")


def _load_chunks(edge_hbm, row, idx_v, wid, n, fill, base, rem, kt):
    pltpu.sync_copy(edge_hbm.at[row].at[pl.ds(wid * base, base)],
                    idx_v.at[pl.ds(0, base)])
    for rr in range(base, kt):
        for t in range(_CHUNK // 16):
            idx_v[rr, pl.ds(16 * t, 16)] = jnp.full((16,), fill, jnp.int32)
    if rem:
        @pl.when(wid < rem)
        def _():
            pltpu.sync_copy(edge_hbm.at[row].at[pl.ds(base * _NW + wid, 1)],
                            idx_v.at[pl.ds(base, 1)])


def _make_deg_kernel(n, base, rem, kt, npad, rpt):
    @functools.partial(
        pl.kernel,
        out_type=jax.ShapeDtypeStruct((_NC, npad, _DEG_W), jnp.float32),
        mesh=_sc_mesh(),
        scratch_types=[
            pltpu.VMEM((kt, _CHUNK), jnp.int32),
            pltpu.VMEM((_CHUNK, _DEG_W), jnp.float32),
            pltpu.VMEM_SHARED((npad, _DEG_W), jnp.float32),
        ],
        compiler_params=pltpu.CompilerParams(use_tc_tiling_on_sc=False),
    )
    def deg_kernel(edge_hbm, ones_hbm, zeros_hbm, out_hbm, dst_v, ones_v, acc):
        c = lax.axis_index("c")
        s = lax.axis_index("s")
        wid = s * _NC + c
        row0 = s * rpt
        _load_chunks(edge_hbm, 1, dst_v, wid, n, n, base, rem, kt)
        pltpu.sync_copy(ones_hbm, ones_v)
        pltpu.sync_copy(zeros_hbm.at[pl.ds(row0, rpt)], acc.at[pl.ds(row0, rpt)])
        plsc.subcore_barrier()

        def body(j, carry):
            pltpu.sync_copy(ones_v, acc.at[dst_v.at[j]], add=True)
            return carry

        lax.fori_loop(0, kt, body, 0)
        plsc.subcore_barrier()
        pltpu.sync_copy(acc.at[pl.ds(row0, rpt)],
                        out_hbm.at[c].at[pl.ds(row0, rpt)])

    return deg_kernel


def _make_agg_kernel(n, base, rem, kt, npad, rpt, d, n_hbm):
    assert kt % 2 == 0 and kt >= 4

    @functools.partial(
        pl.kernel,
        out_type=jax.ShapeDtypeStruct((_NC, npad, d), jnp.float32),
        mesh=_sc_mesh(),
        scratch_types=[
            pltpu.VMEM((kt, _CHUNK), jnp.int32),
            pltpu.VMEM((kt, _CHUNK), jnp.int32),
            pltpu.VMEM((_CHUNK, d), jnp.float32),
            pltpu.VMEM((_CHUNK, d), jnp.float32),
            pltpu.VMEM_SHARED((npad, d), jnp.float32),
            pltpu.VMEM_SHARED((npad, d), jnp.float32),
            pltpu.SemaphoreType.DMA,
            pltpu.SemaphoreType.DMA,
        ],
        compiler_params=pltpu.CompilerParams(use_tc_tiling_on_sc=False),
    )
    def agg_kernel(edge_hbm, table_hbm, zeros_hbm, out_hbm,
                   src_v, dst_v, rows_a, rows_b, acc, table_s, sem_a, sem_b):
        c = lax.axis_index("c")
        s = lax.axis_index("s")
        wid = s * _NC + c
        row0 = s * rpt
        _load_chunks(edge_hbm, 0, src_v, wid, n, 0, base, rem, kt)
        _load_chunks(edge_hbm, 1, dst_v, wid, n, n, base, rem, kt)
        pltpu.sync_copy(zeros_hbm.at[pl.ds(row0, rpt)], acc.at[pl.ds(row0, rpt)])
        pltpu.sync_copy(table_hbm.at[pl.ds(row0, rpt)],
                        table_s.at[pl.ds(row0, rpt)])
        plsc.subcore_barrier()

        use_hbm = s < n_hbm

        def gather(j, buf, sem):
            @pl.when(use_hbm)
            def _():
                pltpu.async_copy(table_hbm.at[src_v.at[j]], buf, sem)

            @pl.when(jnp.logical_not(use_hbm))
            def _():
                pltpu.async_copy(table_s.at[src_v.at[j]], buf, sem)

        def drain(buf, sem):
            pltpu.make_async_copy(table_hbm.at[src_v.at[0]], buf, sem).wait()

        def scatter(j, buf):
            pltpu.sync_copy(buf, acc.at[dst_v.at[j]], add=True)

        gather(0, rows_a, sem_a)
        gather(1, rows_b, sem_b)

        def body(i, carry):
            j0 = 2 * i
            drain(rows_a, sem_a)
            scatter(j0, rows_a)
            gather(j0 + 2, rows_a, sem_a)
            drain(rows_b, sem_b)
            scatter(j0 + 1, rows_b)
            gather(j0 + 3, rows_b, sem_b)
            return carry

        lax.fori_loop(0, kt // 2 - 1, body, 0)
        drain(rows_a, sem_a)
        scatter(kt - 2, rows_a)
        drain(rows_b, sem_b)
        scatter(kt - 1, rows_b)
        plsc.subcore_barrier()
        pltpu.sync_copy(acc.at[pl.ds(row0, rpt)],
                        out_hbm.at[c].at[pl.ds(row0, rpt)])

    return agg_kernel


def _tc1a_body(x_ref, w_ref, praw_ref):
    praw_ref[...] = jnp.dot(x_ref[...], w_ref[...],
                            preferred_element_type=jnp.float32)


def _make_tc1b(n, d_hid, d1):
    def body(praw_ref, dparts_ref, pre_ref, r_ref):
        deg = dparts_ref[0, :n, :1] + dparts_ref[1, :n, :1] + 1.0
        r = lax.rsqrt(deg)
        r_ref[...] = r
        pre_ref[:n, :d_hid] = praw_ref[...] * r
        pre_ref[:n, d_hid:] = jnp.zeros((n, d1 - d_hid), jnp.float32)
    return body


def _make_tc2(n, d_hid, d_out, d2):
    def body(a1_ref, r_ref, w_ref, pre2_ref):
        r = r_ref[...]
        h = jnp.maximum((a1_ref[0, :n, :d_hid] + a1_ref[1, :n, :d_hid]) * r, 0.0)
        p = jnp.dot(h, w_ref[...], preferred_element_type=jnp.float32)
        pre2_ref[:n, :d_out] = p * r
        pre2_ref[:n, d_out:] = jnp.zeros((n, d2 - d_out), jnp.float32)
    return body


def _make_tc3(n, d_out):
    def body(a2_ref, r_ref, out_ref):
        out_ref[...] = (a2_ref[0, :n, :d_out] + a2_ref[1, :n, :d_out]) * r_ref[...]
    return body


def kernel(x, edge_index, W1, W2):
    n, d_feat = x.shape
    e = edge_index.shape[1]
    d_hid = W1.shape[1]
    d_out = W2.shape[1]

    d1 = 24
    d2 = 8
    npad = ((n + 127) // 128) * 128
    rpt = npad // _NS

    if e % _CHUNK:
        pad = _CHUNK - e % _CHUNK
        edge_index = jnp.concatenate(
            [edge_index,
             jnp.concatenate([jnp.zeros((1, pad), jnp.int32),
                              jnp.full((1, pad), n, jnp.int32)])], axis=1)
        e += pad
    nchunks = e // _CHUNK
    base = nchunks // _NW
    rem = nchunks % _NW
    kt = base + 2 - base % 2
    edge3 = edge_index.reshape(2, nchunks, _CHUNK)

    zeros_w = jnp.zeros((npad, _DEG_W), jnp.float32)
    zeros_d1 = jnp.zeros((npad, d1), jnp.float32)
    zeros_d2 = jnp.zeros((npad, d2), jnp.float32)
    ones_c = jnp.ones((_CHUNK, _DEG_W), jnp.float32)

    dparts = _make_deg_kernel(n, base, rem, kt, npad, rpt)(edge3, ones_c, zeros_w)

    praw = pl.pallas_call(
        _tc1a_body,
        out_shape=jax.ShapeDtypeStruct((n, d_hid), jnp.float32),
    )(x, W1)

    pre1s, r = pl.pallas_call(
        _make_tc1b(n, d_hid, d1),
        out_shape=[
            jax.ShapeDtypeStruct((npad, d1), jnp.float32),
            jax.ShapeDtypeStruct((n, 1), jnp.float32),
        ],
    )(praw, dparts)

    a1 = _make_agg_kernel(n, base, rem, kt, npad, rpt, d1, 8)(
        edge3, pre1s, zeros_d1)

    pre2s = pl.pallas_call(
        _make_tc2(n, d_hid, d_out, d2),
        out_shape=jax.ShapeDtypeStruct((npad, d2), jnp.float32),
    )(a1, r, W2)

    a2 = _make_agg_kernel(n, base, rem, kt, npad, rpt, d2, 8)(
        edge3, pre2s, zeros_d2)

    out = pl.pallas_call(
        _make_tc3(n, d_out),
        out_shape=jax.ShapeDtypeStruct((n, d_out), jnp.float32),
    )(a2, r)

    return out


# speedup vs baseline: 1.2928x; 1.0749x over previous
import functools

import jax
import jax.numpy as jnp
from jax import lax
from jax.experimental import pallas as pl
from jax.experimental.pallas import tpu as pltpu
from jax.experimental.pallas import tpu_sc as plsc

_NC = 2
_NS = 16
_NW = _NC * _NS
_CHUNK = 128
_DEG_W = 8


def _sc_mesh():
    return plsc.VectorSubcoreMesh(core_axis_name="c", subcore_axis_name="s")


def _load_chunks(edge_hbm, row, idx_v, wid, n, fill, base, rem, kt):
    pltpu.sync_copy(edge_hbm.at[row].at[pl.ds(wid * base, base)],
                    idx_v.at[pl.ds(0, base)])
    for rr in range(base, kt):
        for t in range(_CHUNK // 16):
            idx_v[rr, pl.ds(16 * t, 16)] = jnp.full((16,), fill, jnp.int32)
    if rem:
        @pl.when(wid < rem)
        def _():
            pltpu.sync_copy(edge_hbm.at[row].at[pl.ds(base * _NW + wid, 1)],
                            idx_v.at[pl.ds(base, 1)])


def _make_deg_kernel(n, base, rem, kt, npad, rpt):
    @functools.partial(
        pl.kernel,
        out_type=jax.ShapeDtypeStruct((_NC, npad, _DEG_W), jnp.float32),
        mesh=_sc_mesh(),
        scratch_types=[
            pltpu.VMEM((kt, _CHUNK), jnp.int32),
            pltpu.VMEM((_CHUNK, _DEG_W), jnp.float32),
            pltpu.VMEM_SHARED((npad, _DEG_W), jnp.float32),
        ],
        compiler_params=pltpu.CompilerParams(use_tc_tiling_on_sc=False),
    )
    def deg_kernel(edge_hbm, ones_hbm, zeros_hbm, out_hbm, dst_v, ones_v, acc):
        c = lax.axis_index("c")
        s = lax.axis_index("s")
        wid = s * _NC + c
        row0 = s * rpt
        _load_chunks(edge_hbm, 1, dst_v, wid, n, n, base, rem, kt)
        pltpu.sync_copy(ones_hbm, ones_v)
        pltpu.sync_copy(zeros_hbm.at[pl.ds(row0, rpt)], acc.at[pl.ds(row0, rpt)])
        plsc.subcore_barrier()

        def body(j, carry):
            pltpu.sync_copy(ones_v, acc.at[dst_v.at[j]], add=True)
            return carry

        lax.fori_loop(0, kt, body, 0)
        plsc.subcore_barrier()
        pltpu.sync_copy(acc.at[pl.ds(row0, rpt)],
                        out_hbm.at[c].at[pl.ds(row0, rpt)])

    return deg_kernel


def _make_agg_kernel(n, base, rem, kt, npad, rpt, d, n_hbm):
    assert kt % 2 == 0 and kt >= 4

    @functools.partial(
        pl.kernel,
        out_type=jax.ShapeDtypeStruct((_NC, npad, d), jnp.float32),
        mesh=_sc_mesh(),
        scratch_types=[
            pltpu.VMEM((kt, _CHUNK), jnp.int32),
            pltpu.VMEM((kt, _CHUNK), jnp.int32),
            pltpu.VMEM((_CHUNK, d), jnp.float32),
            pltpu.VMEM((_CHUNK, d), jnp.float32),
            pltpu.VMEM_SHARED((npad, d), jnp.float32),
            pltpu.VMEM_SHARED((npad, d), jnp.float32),
            pltpu.SemaphoreType.DMA,
            pltpu.SemaphoreType.DMA,
        ],
        compiler_params=pltpu.CompilerParams(use_tc_tiling_on_sc=False),
    )
    def agg_kernel(edge_hbm, table_hbm, zeros_hbm, out_hbm,
                   src_v, dst_v, rows_a, rows_b, acc, table_s, sem_a, sem_b):
        c = lax.axis_index("c")
        s = lax.axis_index("s")
        wid = s * _NC + c
        row0 = s * rpt
        _load_chunks(edge_hbm, 0, src_v, wid, n, 0, base, rem, kt)
        _load_chunks(edge_hbm, 1, dst_v, wid, n, n, base, rem, kt)
        pltpu.sync_copy(zeros_hbm.at[pl.ds(row0, rpt)], acc.at[pl.ds(row0, rpt)])
        pltpu.sync_copy(table_hbm.at[pl.ds(row0, rpt)],
                        table_s.at[pl.ds(row0, rpt)])
        plsc.subcore_barrier()

        use_hbm = s < n_hbm

        def gather(j, buf, sem):
            @pl.when(use_hbm)
            def _():
                pltpu.async_copy(table_hbm.at[src_v.at[j]], buf, sem)

            @pl.when(jnp.logical_not(use_hbm))
            def _():
                pltpu.async_copy(table_s.at[src_v.at[j]], buf, sem)

        def drain(buf, sem):
            pltpu.make_async_copy(table_hbm.at[src_v.at[0]], buf, sem).wait()

        def scatter(j, buf):
            pltpu.sync_copy(buf, acc.at[dst_v.at[j]], add=True)

        gather(0, rows_a, sem_a)
        gather(1, rows_b, sem_b)

        def body(i, carry):
            j0 = 2 * i
            drain(rows_a, sem_a)
            scatter(j0, rows_a)
            gather(j0 + 2, rows_a, sem_a)
            drain(rows_b, sem_b)
            scatter(j0 + 1, rows_b)
            gather(j0 + 3, rows_b, sem_b)
            return carry

        lax.fori_loop(0, kt // 2 - 1, body, 0)
        drain(rows_a, sem_a)
        scatter(kt - 2, rows_a)
        drain(rows_b, sem_b)
        scatter(kt - 1, rows_b)
        plsc.subcore_barrier()
        pltpu.sync_copy(acc.at[pl.ds(row0, rpt)],
                        out_hbm.at[c].at[pl.ds(row0, rpt)])

    return agg_kernel


def _tc1a_body(x_ref, w_ref, praw_ref):
    praw_ref[...] = jnp.dot(x_ref[...], w_ref[...],
                            preferred_element_type=jnp.float32)


def _make_tc1b(n, d_hid, d1):
    def body(praw_ref, dparts_ref, pre_ref, r_ref):
        deg = dparts_ref[0, :n, :1] + dparts_ref[1, :n, :1] + 1.0
        r = lax.rsqrt(deg)
        r_ref[...] = r
        pre_ref[:n, :d_hid] = praw_ref[...] * r
        pre_ref[:n, d_hid:] = jnp.zeros((n, d1 - d_hid), jnp.float32)
    return body


def _make_tc2(n, d_hid, d_out, d2):
    def body(a1_ref, r_ref, w_ref, pre2_ref):
        r = r_ref[...]
        h = jnp.maximum((a1_ref[0, :n, :d_hid] + a1_ref[1, :n, :d_hid]) * r, 0.0)
        p = jnp.dot(h, w_ref[...], preferred_element_type=jnp.float32)
        pre2_ref[:n, :d_out] = p * r
        pre2_ref[:n, d_out:] = jnp.zeros((n, d2 - d_out), jnp.float32)
    return body


def _make_tc3(n, d_out):
    def body(a2_ref, r_ref, out_ref):
        out_ref[...] = (a2_ref[0, :n, :d_out] + a2_ref[1, :n, :d_out]) * r_ref[...]
    return body


def kernel(x, edge_index, W1, W2):
    n, d_feat = x.shape
    e = edge_index.shape[1]
    d_hid = W1.shape[1]
    d_out = W2.shape[1]

    d1 = 24
    d2 = 8
    npad = ((n + 127) // 128) * 128
    rpt = npad // _NS

    if e % _CHUNK:
        pad = _CHUNK - e % _CHUNK
        edge_index = jnp.concatenate(
            [edge_index,
             jnp.concatenate([jnp.zeros((1, pad), jnp.int32),
                              jnp.full((1, pad), n, jnp.int32)])], axis=1)
        e += pad
    nchunks = e // _CHUNK
    base = nchunks // _NW
    rem = nchunks % _NW
    kt = base + 2 - base % 2
    edge3 = edge_index.reshape(2, nchunks, _CHUNK)

    zeros_w = jnp.zeros((npad, _DEG_W), jnp.float32)
    zeros_d1 = jnp.zeros((npad, d1), jnp.float32)
    zeros_d2 = jnp.zeros((npad, d2), jnp.float32)
    ones_c = jnp.ones((_CHUNK, _DEG_W), jnp.float32)

    dparts = _make_deg_kernel(n, base, rem, kt, npad, rpt)(edge3, ones_c, zeros_w)

    praw = pl.pallas_call(
        _tc1a_body,
        out_shape=jax.ShapeDtypeStruct((n, d_hid), jnp.float32),
    )(x, W1)

    pre1s, r = pl.pallas_call(
        _make_tc1b(n, d_hid, d1),
        out_shape=[
            jax.ShapeDtypeStruct((npad, d1), jnp.float32),
            jax.ShapeDtypeStruct((n, 1), jnp.float32),
        ],
    )(praw, dparts)

    a1 = _make_agg_kernel(n, base, rem, kt, npad, rpt, d1, 6)(
        edge3, pre1s, zeros_d1)

    pre2s = pl.pallas_call(
        _make_tc2(n, d_hid, d_out, d2),
        out_shape=jax.ShapeDtypeStruct((npad, d2), jnp.float32),
    )(a1, r, W2)

    a2 = _make_agg_kernel(n, base, rem, kt, npad, rpt, d2, 4)(
        edge3, pre2s, zeros_d2)

    out = pl.pallas_call(
        _make_tc3(n, d_out),
        out_shape=jax.ShapeDtypeStruct((n, d_out), jnp.float32),
    )(a2, r)

    return out


# speedup vs baseline: 1.3267x; 1.0262x over previous
import functools

import jax
import jax.numpy as jnp
from jax import lax
from jax.experimental import pallas as pl
from jax.experimental.pallas import tpu as pltpu
from jax.experimental.pallas import tpu_sc as plsc

_NC = 2
_NS = 16
_NW = _NC * _NS
_CHUNK = 128
_DEG_W = 8


def _sc_mesh():
    return plsc.VectorSubcoreMesh(core_axis_name="c", subcore_axis_name="s")


def _load_chunks(edge_hbm, row, idx_v, wid, n, fill, base, rem, kt):
    pltpu.sync_copy(edge_hbm.at[row].at[pl.ds(wid * base, base)],
                    idx_v.at[pl.ds(0, base)])
    for rr in range(base, kt):
        for t in range(_CHUNK // 16):
            idx_v[rr, pl.ds(16 * t, 16)] = jnp.full((16,), fill, jnp.int32)
    if rem:
        @pl.when(wid < rem)
        def _():
            pltpu.sync_copy(edge_hbm.at[row].at[pl.ds(base * _NW + wid, 1)],
                            idx_v.at[pl.ds(base, 1)])


def _make_deg_kernel(n, base, rem, kt, npad, rpt):
    @functools.partial(
        pl.kernel,
        out_type=jax.ShapeDtypeStruct((_NC, npad, _DEG_W), jnp.float32),
        mesh=_sc_mesh(),
        scratch_types=[
            pltpu.VMEM((kt, _CHUNK), jnp.int32),
            pltpu.VMEM((_CHUNK, _DEG_W), jnp.float32),
            pltpu.VMEM_SHARED((npad, _DEG_W), jnp.float32),
        ],
        compiler_params=pltpu.CompilerParams(use_tc_tiling_on_sc=False),
    )
    def deg_kernel(edge_hbm, ones_hbm, zeros_hbm, out_hbm, dst_v, ones_v, acc):
        c = lax.axis_index("c")
        s = lax.axis_index("s")
        wid = s * _NC + c
        row0 = s * rpt
        _load_chunks(edge_hbm, 1, dst_v, wid, n, n, base, rem, kt)
        pltpu.sync_copy(ones_hbm, ones_v)
        pltpu.sync_copy(zeros_hbm.at[pl.ds(row0, rpt)], acc.at[pl.ds(row0, rpt)])
        plsc.subcore_barrier()

        def body(j, carry):
            pltpu.sync_copy(ones_v, acc.at[dst_v.at[j]], add=True)
            return carry

        lax.fori_loop(0, kt, body, 0)
        plsc.subcore_barrier()
        pltpu.sync_copy(acc.at[pl.ds(row0, rpt)],
                        out_hbm.at[c].at[pl.ds(row0, rpt)])

    return deg_kernel


def _make_agg_kernel(n, base, rem, kt, npad, rpt, d, n_hbm):
    assert kt % 2 == 0 and kt >= 4

    @functools.partial(
        pl.kernel,
        out_type=jax.ShapeDtypeStruct((_NC, npad, d), jnp.float32),
        mesh=_sc_mesh(),
        scratch_types=[
            pltpu.VMEM((kt, _CHUNK), jnp.int32),
            pltpu.VMEM((kt, _CHUNK), jnp.int32),
            pltpu.VMEM((_CHUNK, d), jnp.float32),
            pltpu.VMEM((_CHUNK, d), jnp.float32),
            pltpu.VMEM_SHARED((npad, d), jnp.float32),
            pltpu.VMEM_SHARED((npad, d), jnp.float32),
            pltpu.SemaphoreType.DMA,
            pltpu.SemaphoreType.DMA,
        ],
        compiler_params=pltpu.CompilerParams(use_tc_tiling_on_sc=False),
    )
    def agg_kernel(edge_hbm, table_hbm, zeros_hbm, out_hbm,
                   src_v, dst_v, rows_a, rows_b, acc, table_s, sem_a, sem_b):
        c = lax.axis_index("c")
        s = lax.axis_index("s")
        wid = s * _NC + c
        row0 = s * rpt
        _load_chunks(edge_hbm, 0, src_v, wid, n, 0, base, rem, kt)
        _load_chunks(edge_hbm, 1, dst_v, wid, n, n, base, rem, kt)
        pltpu.sync_copy(zeros_hbm.at[pl.ds(row0, rpt)], acc.at[pl.ds(row0, rpt)])
        pltpu.sync_copy(table_hbm.at[pl.ds(row0, rpt)],
                        table_s.at[pl.ds(row0, rpt)])
        plsc.subcore_barrier()

        use_hbm = s < n_hbm

        def gather(j, buf, sem):
            @pl.when(use_hbm)
            def _():
                pltpu.async_copy(table_hbm.at[src_v.at[j]], buf, sem)

            @pl.when(jnp.logical_not(use_hbm))
            def _():
                pltpu.async_copy(table_s.at[src_v.at[j]], buf, sem)

        def drain(buf, sem):
            pltpu.make_async_copy(table_hbm.at[src_v.at[0]], buf, sem).wait()

        def scatter(j, buf):
            pltpu.sync_copy(buf, acc.at[dst_v.at[j]], add=True)

        gather(0, rows_a, sem_a)
        gather(1, rows_b, sem_b)

        def body(i, carry):
            j0 = 2 * i
            drain(rows_a, sem_a)
            scatter(j0, rows_a)
            gather(j0 + 2, rows_a, sem_a)
            drain(rows_b, sem_b)
            scatter(j0 + 1, rows_b)
            gather(j0 + 3, rows_b, sem_b)
            return carry

        lax.fori_loop(0, kt // 2 - 1, body, 0)
        drain(rows_a, sem_a)
        scatter(kt - 2, rows_a)
        drain(rows_b, sem_b)
        scatter(kt - 1, rows_b)
        plsc.subcore_barrier()
        pltpu.sync_copy(acc.at[pl.ds(row0, rpt)],
                        out_hbm.at[c].at[pl.ds(row0, rpt)])

    return agg_kernel


def _tc1a_body(x_ref, w_ref, praw_ref):
    praw_ref[...] = jnp.dot(x_ref[...], w_ref[...],
                            preferred_element_type=jnp.float32)


def _make_tc1b(n, d_hid, d1):
    def body(praw_ref, dparts_ref, pre_ref, r_ref):
        deg = dparts_ref[0, :n, :1] + dparts_ref[1, :n, :1] + 1.0
        r = lax.rsqrt(deg)
        r_ref[...] = r
        pre_ref[:n, :d_hid] = praw_ref[...] * r
        pre_ref[:n, d_hid:] = jnp.zeros((n, d1 - d_hid), jnp.float32)
    return body


def _make_tc2(n, d_hid, d_out, d2):
    def body(a1_ref, r_ref, w_ref, pre2_ref):
        r = r_ref[...]
        h = jnp.maximum((a1_ref[0, :n, :d_hid] + a1_ref[1, :n, :d_hid]) * r, 0.0)
        p = jnp.dot(h, w_ref[...], preferred_element_type=jnp.float32)
        pre2_ref[:n, :d_out] = p * r
        pre2_ref[:n, d_out:] = jnp.zeros((n, d2 - d_out), jnp.float32)
    return body


def _make_tc3(n, d_out):
    def body(a2_ref, r_ref, out_ref):
        out_ref[...] = (a2_ref[0, :n, :d_out] + a2_ref[1, :n, :d_out]) * r_ref[...]
    return body


def kernel(x, edge_index, W1, W2):
    n, d_feat = x.shape
    e = edge_index.shape[1]
    d_hid = W1.shape[1]
    d_out = W2.shape[1]

    d1 = 24
    d2 = 8
    npad = ((n + 127) // 128) * 128
    rpt = npad // _NS

    if e % _CHUNK:
        pad = _CHUNK - e % _CHUNK
        edge_index = jnp.concatenate(
            [edge_index,
             jnp.concatenate([jnp.zeros((1, pad), jnp.int32),
                              jnp.full((1, pad), n, jnp.int32)])], axis=1)
        e += pad
    nchunks = e // _CHUNK
    base = nchunks // _NW
    rem = nchunks % _NW
    kt = base + 2 - base % 2
    edge3 = edge_index.reshape(2, nchunks, _CHUNK)

    zeros_w = jnp.zeros((npad, _DEG_W), jnp.float32)
    zeros_d1 = jnp.zeros((npad, d1), jnp.float32)
    zeros_d2 = jnp.zeros((npad, d2), jnp.float32)
    ones_c = jnp.ones((_CHUNK, _DEG_W), jnp.float32)

    dparts = _make_deg_kernel(n, base, rem, kt, npad, rpt)(edge3, ones_c, zeros_w)

    praw = pl.pallas_call(
        _tc1a_body,
        out_shape=jax.ShapeDtypeStruct((n, d_hid), jnp.float32),
    )(x, W1)

    pre1s, r = pl.pallas_call(
        _make_tc1b(n, d_hid, d1),
        out_shape=[
            jax.ShapeDtypeStruct((npad, d1), jnp.float32),
            jax.ShapeDtypeStruct((n, 1), jnp.float32),
        ],
    )(praw, dparts)

    a1 = _make_agg_kernel(n, base, rem, kt, npad, rpt, d1, 5)(
        edge3, pre1s, zeros_d1)

    pre2s = pl.pallas_call(
        _make_tc2(n, d_hid, d_out, d2),
        out_shape=jax.ShapeDtypeStruct((npad, d2), jnp.float32),
    )(a1, r, W2)

    a2 = _make_agg_kernel(n, base, rem, kt, npad, rpt, d2, 3)(
        edge3, pre2s, zeros_d2)

    out = pl.pallas_call(
        _make_tc3(n, d_out),
        out_shape=jax.ShapeDtypeStruct((n, d_out), jnp.float32),
    )(a2, r)

    return out


# speedup vs baseline: 1.3605x; 1.0255x over previous
import functools

import jax
import jax.numpy as jnp
from jax import lax
from jax.experimental import pallas as pl
from jax.experimental.pallas import tpu as pltpu
from jax.experimental.pallas import tpu_sc as plsc

_NC = 2
_NS = 16
_NW = _NC * _NS
_CHUNK = 128
_DEG_W = 8


def _sc_mesh():
    return plsc.VectorSubcoreMesh(core_axis_name="c", subcore_axis_name="s")


def _load_chunks(edge_hbm, row, idx_v, wid, n, fill, base, rem, kt):
    pltpu.sync_copy(edge_hbm.at[row].at[pl.ds(wid * base, base)],
                    idx_v.at[pl.ds(0, base)])
    for rr in range(base, kt):
        for t in range(_CHUNK // 16):
            idx_v[rr, pl.ds(16 * t, 16)] = jnp.full((16,), fill, jnp.int32)
    if rem:
        @pl.when(wid < rem)
        def _():
            pltpu.sync_copy(edge_hbm.at[row].at[pl.ds(base * _NW + wid, 1)],
                            idx_v.at[pl.ds(base, 1)])


def _make_deg_kernel(n, base, rem, kt, npad, rpt):
    @functools.partial(
        pl.kernel,
        out_type=jax.ShapeDtypeStruct((_NC, npad, _DEG_W), jnp.float32),
        mesh=_sc_mesh(),
        scratch_types=[
            pltpu.VMEM((kt, _CHUNK), jnp.int32),
            pltpu.VMEM((_CHUNK, _DEG_W), jnp.float32),
            pltpu.VMEM_SHARED((npad, _DEG_W), jnp.float32),
        ],
        compiler_params=pltpu.CompilerParams(use_tc_tiling_on_sc=False),
    )
    def deg_kernel(edge_hbm, ones_hbm, zeros_hbm, out_hbm, dst_v, ones_v, acc):
        c = lax.axis_index("c")
        s = lax.axis_index("s")
        wid = s * _NC + c
        row0 = s * rpt
        _load_chunks(edge_hbm, 1, dst_v, wid, n, n, base, rem, kt)
        pltpu.sync_copy(ones_hbm, ones_v)
        pltpu.sync_copy(zeros_hbm.at[pl.ds(row0, rpt)], acc.at[pl.ds(row0, rpt)])
        plsc.subcore_barrier()

        def body(j, carry):
            pltpu.sync_copy(ones_v, acc.at[dst_v.at[j]], add=True)
            return carry

        lax.fori_loop(0, kt, body, 0)
        plsc.subcore_barrier()
        pltpu.sync_copy(acc.at[pl.ds(row0, rpt)],
                        out_hbm.at[c].at[pl.ds(row0, rpt)])

    return deg_kernel


def _make_agg_kernel(n, base, rem, kt, npad, rpt, d, n_hbm):
    assert kt % 2 == 0 and kt >= 4

    @functools.partial(
        pl.kernel,
        out_type=jax.ShapeDtypeStruct((_NC, npad, d), jnp.float32),
        mesh=_sc_mesh(),
        scratch_types=[
            pltpu.VMEM((kt, _CHUNK), jnp.int32),
            pltpu.VMEM((kt, _CHUNK), jnp.int32),
            pltpu.VMEM((_CHUNK, d), jnp.float32),
            pltpu.VMEM((_CHUNK, d), jnp.float32),
            pltpu.VMEM_SHARED((npad, d), jnp.float32),
            pltpu.VMEM_SHARED((npad, d), jnp.float32),
            pltpu.SemaphoreType.DMA,
            pltpu.SemaphoreType.DMA,
        ],
        compiler_params=pltpu.CompilerParams(use_tc_tiling_on_sc=False),
    )
    def agg_kernel(edge_hbm, table_hbm, zeros_hbm, out_hbm,
                   src_v, dst_v, rows_a, rows_b, acc, table_s, sem_a, sem_b):
        c = lax.axis_index("c")
        s = lax.axis_index("s")
        wid = s * _NC + c
        row0 = s * rpt
        _load_chunks(edge_hbm, 0, src_v, wid, n, 0, base, rem, kt)
        _load_chunks(edge_hbm, 1, dst_v, wid, n, n, base, rem, kt)
        pltpu.sync_copy(zeros_hbm.at[pl.ds(row0, rpt)], acc.at[pl.ds(row0, rpt)])
        pltpu.sync_copy(table_hbm.at[pl.ds(row0, rpt)],
                        table_s.at[pl.ds(row0, rpt)])
        plsc.subcore_barrier()

        use_hbm = s < n_hbm

        def gather(j, buf, sem):
            @pl.when(use_hbm)
            def _():
                pltpu.async_copy(table_hbm.at[src_v.at[j]], buf, sem)

            @pl.when(jnp.logical_not(use_hbm))
            def _():
                pltpu.async_copy(table_s.at[src_v.at[j]], buf, sem)

        def drain(buf, sem):
            pltpu.make_async_copy(table_hbm.at[src_v.at[0]], buf, sem).wait()

        def scatter(j, buf):
            pltpu.sync_copy(buf, acc.at[dst_v.at[j]], add=True)

        gather(0, rows_a, sem_a)
        gather(1, rows_b, sem_b)

        def body(i, carry):
            j0 = 2 * i
            drain(rows_a, sem_a)
            scatter(j0, rows_a)
            gather(j0 + 2, rows_a, sem_a)
            drain(rows_b, sem_b)
            scatter(j0 + 1, rows_b)
            gather(j0 + 3, rows_b, sem_b)
            return carry

        lax.fori_loop(0, kt // 2 - 1, body, 0)
        drain(rows_a, sem_a)
        scatter(kt - 2, rows_a)
        drain(rows_b, sem_b)
        scatter(kt - 1, rows_b)
        plsc.subcore_barrier()
        pltpu.sync_copy(acc.at[pl.ds(row0, rpt)],
                        out_hbm.at[c].at[pl.ds(row0, rpt)])

    return agg_kernel


def _tc1a_body(x_ref, w_ref, praw_ref):
    praw_ref[...] = jnp.dot(x_ref[...], w_ref[...],
                            preferred_element_type=jnp.float32)


def _make_tc1b(n, d_hid, d1):
    def body(praw_ref, dparts_ref, pre_ref, r_ref):
        deg = dparts_ref[0, :n, :1] + dparts_ref[1, :n, :1] + 1.0
        r = lax.rsqrt(deg)
        r_ref[...] = r
        pre_ref[:n, :d_hid] = praw_ref[...] * r
        pre_ref[:n, d_hid:] = jnp.zeros((n, d1 - d_hid), jnp.float32)
    return body


def _make_tc2(n, d_hid, d_out, d2):
    def body(a1_ref, r_ref, w_ref, pre2_ref):
        r = r_ref[...]
        h = jnp.maximum((a1_ref[0, :n, :d_hid] + a1_ref[1, :n, :d_hid]) * r, 0.0)
        p = jnp.dot(h, w_ref[...], preferred_element_type=jnp.float32)
        pre2_ref[:n, :d_out] = p * r
        pre2_ref[:n, d_out:] = jnp.zeros((n, d2 - d_out), jnp.float32)
    return body


def _make_tc3(n, d_out):
    def body(a2_ref, r_ref, out_ref):
        out_ref[...] = (a2_ref[0, :n, :d_out] + a2_ref[1, :n, :d_out]) * r_ref[...]
    return body


def kernel(x, edge_index, W1, W2):
    n, d_feat = x.shape
    e = edge_index.shape[1]
    d_hid = W1.shape[1]
    d_out = W2.shape[1]

    d1 = 24
    d2 = 8
    npad = ((n + 127) // 128) * 128
    rpt = npad // _NS

    if e % _CHUNK:
        pad = _CHUNK - e % _CHUNK
        edge_index = jnp.concatenate(
            [edge_index,
             jnp.concatenate([jnp.zeros((1, pad), jnp.int32),
                              jnp.full((1, pad), n, jnp.int32)])], axis=1)
        e += pad
    nchunks = e // _CHUNK
    base = nchunks // _NW
    rem = nchunks % _NW
    kt = base + 2 - base % 2
    edge3 = edge_index.reshape(2, nchunks, _CHUNK)

    zeros_w = jnp.zeros((npad, _DEG_W), jnp.float32)
    zeros_d1 = jnp.zeros((npad, d1), jnp.float32)
    zeros_d2 = jnp.zeros((npad, d2), jnp.float32)
    ones_c = jnp.ones((_CHUNK, _DEG_W), jnp.float32)

    dparts = _make_deg_kernel(n, base, rem, kt, npad, rpt)(edge3, ones_c, zeros_w)

    praw = pl.pallas_call(
        _tc1a_body,
        out_shape=jax.ShapeDtypeStruct((n, d_hid), jnp.float32),
    )(x, W1)

    pre1s, r = pl.pallas_call(
        _make_tc1b(n, d_hid, d1),
        out_shape=[
            jax.ShapeDtypeStruct((npad, d1), jnp.float32),
            jax.ShapeDtypeStruct((n, 1), jnp.float32),
        ],
    )(praw, dparts)

    a1 = _make_agg_kernel(n, base, rem, kt, npad, rpt, d1, 4)(
        edge3, pre1s, zeros_d1)

    pre2s = pl.pallas_call(
        _make_tc2(n, d_hid, d_out, d2),
        out_shape=jax.ShapeDtypeStruct((npad, d2), jnp.float32),
    )(a1, r, W2)

    a2 = _make_agg_kernel(n, base, rem, kt, npad, rpt, d2, 2)(
        edge3, pre2s, zeros_d2)

    out = pl.pallas_call(
        _make_tc3(n, d_out),
        out_shape=jax.ShapeDtypeStruct((n, d_out), jnp.float32),
    )(a2, r)

    return out
